# baseline (speedup 1.0000x reference)
import functools

import numpy as np
import jax
import jax.numpy as jnp
from jax import lax
from jax.experimental import pallas as pl
from jax.experimental.pallas import tpu as pltpu
from jax.experimental.pallas import tpu_sc as plsc

N = 10000
E = 320000
F = 128
NEG = 0.2
NP = 10240
NC, NS, L = 2, 16, 16
NW = NC * NS
CH = 96
GCH = 108
EW = GCH * CH
EP = NW * EW
TRI = GCH // 3
C10 = float(np.log(1e-10))
RPT = NP // NS

_mesh = plsc.VectorSubcoreMesh(core_axis_name="c", subcore_axis_name="s")
_sc_params = pltpu.CompilerParams(needs_layout_passes=False)


def _proj_body(feat_ref, w_ref, al_ref, ar_ref, h_ref, sc_ref):
    x = feat_ref[...]
    w = w_ref[...]
    h = lax.dot_general(x, w, (((1,), (1,)), ((), ())),
                        preferred_element_type=jnp.float32)
    h_ref[...] = h
    el = lax.dot_general(al_ref[...], h, (((1,), (1,)), ((), ())),
                         preferred_element_type=jnp.float32)
    er = lax.dot_general(ar_ref[...], h, (((1,), (1,)), ((), ())),
                         preferred_element_type=jnp.float32)
    sc_ref[...] = jnp.concatenate([el, er], axis=0)


_BA = 1024
_proj = pl.pallas_call(
    _proj_body,
    grid=(NP // _BA,),
    in_specs=[pl.BlockSpec((_BA, 128), lambda i: (i, 0)),
              pl.BlockSpec((128, 128), lambda i: (0, 0)),
              pl.BlockSpec((1, 128), lambda i: (0, 0)),
              pl.BlockSpec((1, 128), lambda i: (0, 0))],
    out_specs=[pl.BlockSpec((_BA, 128), lambda i: (i, 0)),
               pl.BlockSpec((2, _BA), lambda i: (0, i))],
    out_shape=[jax.ShapeDtypeStruct((NP, 128), jnp.float32),
               jax.ShapeDtypeStruct((2, NP), jnp.float32)],
)


@functools.partial(
    pl.kernel,
    out_type=jax.ShapeDtypeStruct((NW, 4, NP), jnp.float32),
    mesh=_mesh,
    scratch_types=[
        pltpu.VMEM((2, NP), jnp.float32),
        pltpu.VMEM((NP,), jnp.float32),
        pltpu.VMEM((NP,), jnp.float32),
        pltpu.VMEM((NP,), jnp.float32),
        pltpu.VMEM((NP,), jnp.float32),
        pltpu.VMEM((EW,), jnp.int32),
        pltpu.VMEM((EW,), jnp.int32),
    ],
    compiler_params=_sc_params,
)
def _edge_pass1(sc_hbm, src_hbm, dst_hbm, out_hbm,
                nv, a_ss, a_sd, a_do, a_di, se, de):
    cid = lax.axis_index("c")
    sid = lax.axis_index("s")
    wid = cid * NS + sid
    pltpu.sync_copy(sc_hbm, nv)
    pltpu.sync_copy(src_hbm.at[pl.ds(wid * EW, EW)], se)
    pltpu.sync_copy(dst_hbm.at[pl.ds(wid * EW, EW)], de)
    zf = jnp.zeros((16,), jnp.float32)

    def zbody(i, _):
        a_ss[pl.ds(i * 16, 16)] = zf
        a_sd[pl.ds(i * 16, 16)] = zf
        a_do[pl.ds(i * 16, 16)] = zf
        a_di[pl.ds(i * 16, 16)] = zf
        return 0

    lax.fori_loop(0, NP // 16, zbody, 0)
    c0 = jnp.zeros((16,), jnp.int32)
    c1 = jnp.ones((16,), jnp.int32)
    onef = jnp.ones((16,), jnp.float32)

    def gbody(g, _):
        base = g * CH
        for j in range(CH // 16):
            s = se[pl.ds(base + j * 16, 16)]
            d = de[pl.ds(base + j * 16, 16)]
            elv = plsc.load_gather(nv, [c0, s])
            erv = plsc.load_gather(nv, [c1, d])
            t = elv + erv
            ex = jnp.exp(jnp.where(t >= 0.0, t, t * NEG))
            plsc.addupdate_scatter(a_ss, [s], ex)
            plsc.addupdate_scatter(a_sd, [d], ex)
            plsc.addupdate_scatter(a_do, [s], onef)
            plsc.addupdate_scatter(a_di, [d], onef)
        return 0

    lax.fori_loop(0, GCH, gbody, 0)
    pltpu.sync_copy(a_ss, out_hbm.at[wid, 0])
    pltpu.sync_copy(a_sd, out_hbm.at[wid, 1])
    pltpu.sync_copy(a_do, out_hbm.at[wid, 2])
    pltpu.sync_copy(a_di, out_hbm.at[wid, 3])


def _nodecalc_body(p_ref, sc_ref, sig_ref, o_ref):
    p = p_ref[...]
    s = jnp.sum(p, axis=0)
    el = sc_ref[0, :]
    er = sc_ref[1, :]
    ls_src = jnp.log(jnp.maximum(s[0], 1e-38))
    ls_dst = jnp.log(jnp.maximum(s[1], 1e-38))
    lo = -0.5 * jnp.log(jnp.maximum(s[2], 1.0))
    li = 0.5 * jnp.log(jnp.maximum(s[3], 1.0))
    sg = 1.0 / (1.0 + jnp.exp(-sig_ref[0]))
    sgr = jnp.full_like(el, sg)
    o_ref[...] = jnp.stack(
        [el, er, ls_src, ls_dst, lo, li, sgr, jnp.zeros_like(el)], axis=0)


_BC = 2048
_nodecalc = pl.pallas_call(
    _nodecalc_body,
    grid=(NP // _BC,),
    in_specs=[pl.BlockSpec((NW, 4, _BC), lambda i: (0, 0, i)),
              pl.BlockSpec((2, _BC), lambda i: (0, i)),
              pl.BlockSpec(memory_space=pltpu.SMEM)],
    out_specs=pl.BlockSpec((8, _BC), lambda i: (0, i)),
    out_shape=jax.ShapeDtypeStruct((8, NP), jnp.float32),
)


@functools.partial(
    pl.kernel,
    out_type=jax.ShapeDtypeStruct((EP,), jnp.float32),
    mesh=_mesh,
    scratch_types=[
        pltpu.VMEM((8, NP), jnp.float32),
        pltpu.VMEM((EW,), jnp.int32),
        pltpu.VMEM((EW,), jnp.int32),
        pltpu.VMEM((EW,), jnp.float32),
    ],
    compiler_params=_sc_params,
)
def _edge_pass2(nsc_hbm, src_hbm, dst_hbm, out_hbm, nv, se, de, wl):
    cid = lax.axis_index("c")
    sid = lax.axis_index("s")
    wid = cid * NS + sid
    pltpu.sync_copy(nsc_hbm, nv)
    pltpu.sync_copy(src_hbm.at[pl.ds(wid * EW, EW)], se)
    pltpu.sync_copy(dst_hbm.at[pl.ds(wid * EW, EW)], de)
    cs = [jnp.full((16,), k, jnp.int32) for k in range(6)]
    sgv = nv[6, pl.ds(0, 16)]

    def gbody(g, _):
        base = g * CH
        for j in range(CH // 16):
            o = base + j * 16
            s = se[pl.ds(o, 16)]
            d = de[pl.ds(o, 16)]
            elv = plsc.load_gather(nv, [cs[0], s])
            erv = plsc.load_gather(nv, [cs[1], d])
            lss = plsc.load_gather(nv, [cs[2], s])
            lsd = plsc.load_gather(nv, [cs[3], d])
            lov = plsc.load_gather(nv, [cs[4], s])
            liv = plsc.load_gather(nv, [cs[5], d])
            t = elv + erv
            e = jnp.where(t >= 0.0, t, t * NEG)
            las = jnp.maximum(e - lss, C10)
            lad = jnp.maximum(e - lsd, C10)
            wl[pl.ds(o, 16)] = jnp.exp(
                sgv * lad + (1.0 - sgv) * las + lov + liv)
        return 0

    lax.fori_loop(0, GCH, gbody, 0)
    pltpu.sync_copy(wl, out_hbm.at[pl.ds(wid * EW, EW)])


@functools.partial(
    pl.kernel,
    out_type=jax.ShapeDtypeStruct((NC, NP, 128), jnp.float32),
    mesh=_mesh,
    scratch_types=[
        pltpu.VMEM((EW,), jnp.int32),
        pltpu.VMEM((CH,), jnp.int32),
        pltpu.VMEM((CH,), jnp.int32),
        pltpu.VMEM((CH,), jnp.int32),
        pltpu.VMEM((CH,), jnp.float32),
        pltpu.VMEM((CH,), jnp.float32),
        pltpu.VMEM((CH,), jnp.float32),
        pltpu.VMEM((CH, 128), jnp.float32),
        pltpu.VMEM((CH, 128), jnp.float32),
        pltpu.VMEM((CH, 128), jnp.float32),
        pltpu.VMEM_SHARED((NP, 128), jnp.float32),
        pltpu.SemaphoreType.DMA,
        pltpu.SemaphoreType.DMA,
        pltpu.SemaphoreType.DMA,
        pltpu.SemaphoreType.DMA,
        pltpu.SemaphoreType.DMA,
        pltpu.SemaphoreType.DMA,
        pltpu.SemaphoreType.DMA,
        pltpu.SemaphoreType.DMA,
        pltpu.SemaphoreType.DMA,
    ],
    compiler_params=_sc_params,
)
def _hop(x_hbm, w_hbm, src_hbm, dst_hbm, out_hbm,
         se, db0, db1, db2, wb0, wb1, wb2, rows0, rows1, rows2, acc,
         semg0, semg1, semg2, semi0, semi1, semi2, sems0, sems1, sems2):
    cid = lax.axis_index("c")
    sid = lax.axis_index("s")
    wid = cid * NS + sid
    pltpu.sync_copy(src_hbm.at[pl.ds(wid * EW, EW)], se)
    zf = jnp.zeros((16,), jnp.float32)

    def zrow(r, _):
        for j in range(8):
            rows0[r, pl.ds(j * 16, 16)] = zf
        return 0

    lax.fori_loop(0, CH, zrow, 0)
    base0 = sid * RPT
    for b in range(RPT // CH):
        pltpu.sync_copy(rows0, acc.at[pl.ds(base0 + b * CH, CH)])
    rem = RPT - (RPT // CH) * CH
    if rem:
        pltpu.sync_copy(rows0.at[pl.ds(0, rem)],
                        acc.at[pl.ds(base0 + (RPT // CH) * CH, rem)])
    plsc.subcore_barrier()

    rows = (rows0, rows1, rows2)
    dbs = (db0, db1, db2)
    wbs = (wb0, wb1, wb2)
    semg = (semg0, semg1, semg2)
    semi = (semi0, semi1, semi2)
    sems = (sems0, sems1, sems2)

    def prefetch(g, slot):
        pltpu.async_copy(w_hbm.at[pl.ds(wid * EW + g * CH, CH)], wbs[slot],
                         semi[slot])
        pltpu.async_copy(dst_hbm.at[pl.ds(wid * EW + g * CH, CH)], dbs[slot],
                         semi[slot])
        pltpu.async_copy(x_hbm.at[se.at[pl.ds(g * CH, CH)]], rows[slot],
                         semg[slot])

    def wait_scatter(slot):
        pltpu.make_async_copy(rows[slot], acc.at[dbs[slot]],
                              sems[slot]).wait()

    def process(g, slot):
        rb = rows[slot]
        wb = wbs[slot]
        pltpu.make_async_copy(w_hbm.at[pl.ds(wid * EW + g * CH, CH)], wb,
                              semi[slot]).wait()
        pltpu.make_async_copy(dst_hbm.at[pl.ds(wid * EW + g * CH, CH)],
                              dbs[slot], semi[slot]).wait()
        pltpu.make_async_copy(x_hbm.at[se.at[pl.ds(g * CH, CH)]], rb,
                              semg[slot]).wait()

        @plsc.parallel_loop(0, CH // 16, unroll=2)
        def sgrp(q):
            wv16 = wb[pl.ds(q * 16, 16)]
            base = q * 16
            for r in range(16):
                wvr = jnp.broadcast_to(wv16[r], (16,))
                for j in range(8):
                    rb[base + r, pl.ds(j * 16, 16)] = (
                        rb[base + r, pl.ds(j * 16, 16)] * wvr)
        pltpu.async_copy(rb, acc.at[dbs[slot]], sems[slot], add=True)

    prefetch(0, 0)
    prefetch(1, 1)

    def rbody(r, _):
        for b in range(3):
            g = 3 * r + b
            slot = b
            slot2 = (b + 2) % 3
            if b == 0:
                @pl.when(r >= 1)
                def _():
                    wait_scatter(slot2)
                prefetch(g + 2, slot2)
            else:
                @pl.when(r < TRI - 1)
                def _():
                    wait_scatter(slot2)
                    prefetch(g + 2, slot2)
            process(g, slot)
        return 0

    lax.fori_loop(0, TRI, rbody, 0)
    for s in range(3):
        wait_scatter(s)
    plsc.subcore_barrier()
    for b in range(RPT // 128):
        r0 = sid * RPT + b * 128
        pltpu.sync_copy(acc.at[pl.ds(r0, 128)],
                        out_hbm.at[cid, pl.ds(r0, 128)])


def _comb_body(p_ref, o_ref):
    o_ref[...] = p_ref[0] + p_ref[1]


_BB = 1024
_comb = pl.pallas_call(
    _comb_body,
    grid=(NP // _BB,),
    in_specs=[pl.BlockSpec((NC, _BB, 128), lambda i: (0, i, 0))],
    out_specs=pl.BlockSpec((_BB, 128), lambda i: (i, 0)),
    out_shape=jax.ShapeDtypeStruct((NP, 128), jnp.float32),
)


def _final_body(h_ref, x1_ref, x2_ref, p3_ref, hl_ref, hr_ref, o_ref):
    h = h_ref[...]
    x1 = x1_ref[...]
    x2 = x2_ref[...]
    x3 = p3_ref[0] + p3_ref[1]
    hl = hl_ref[...]
    hr = hr_ref[...]
    al = jnp.sum(h * hl, axis=1, keepdims=True)
    xs = (h, x1, x2, x3)
    ls = []
    for x in xs:
        v = al + jnp.sum(x * hr, axis=1, keepdims=True)
        ls.append(jnp.where(v >= 0.0, v, v * NEG))
    m = jnp.maximum(jnp.maximum(ls[0], ls[1]), jnp.maximum(ls[2], ls[3]))
    es = [jnp.exp(v - m) for v in ls]
    tot = es[0] + es[1] + es[2] + es[3]
    o_ref[...] = (h * es[0] + x1 * es[1] + x2 * es[2] + x3 * es[3]) / tot


_final = pl.pallas_call(
    _final_body,
    grid=(NP // _BA,),
    in_specs=[pl.BlockSpec((_BA, 128), lambda i: (i, 0)),
              pl.BlockSpec((_BA, 128), lambda i: (i, 0)),
              pl.BlockSpec((_BA, 128), lambda i: (i, 0)),
              pl.BlockSpec((NC, _BA, 128), lambda i: (0, i, 0)),
              pl.BlockSpec((1, 128), lambda i: (0, 0)),
              pl.BlockSpec((1, 128), lambda i: (0, 0))],
    out_specs=pl.BlockSpec((_BA, 128), lambda i: (i, 0)),
    out_shape=jax.ShapeDtypeStruct((NP, 128), jnp.float32),
)


def kernel(feat, W_fc, attn_l, attn_r, hop_attn_l, hop_attn_r, sigma,
           edge_index):
    feat_p = jnp.pad(feat, ((0, NP - N), (0, 0)))
    al = attn_l.reshape(1, F)
    ar = attn_r.reshape(1, F)
    hl = hop_attn_l.reshape(1, F)
    hr = hop_attn_r.reshape(1, F)
    padn = EP - E
    pad_idx = N + (jnp.arange(padn, dtype=jnp.int32) % (NP - N))
    srcp = jnp.concatenate([edge_index[0], pad_idx])
    dstp = jnp.concatenate([edge_index[1], pad_idx])

    h_pad, sc1 = _proj(feat_p, W_fc, al, ar)
    part1 = _edge_pass1(sc1, srcp, dstp)
    nsc = _nodecalc(part1, sc1, sigma)
    wp = _edge_pass2(nsc, srcp, dstp)
    p1 = _hop(h_pad, wp, srcp, dstp)
    x1 = _comb(p1)
    p2 = _hop(x1, wp, srcp, dstp)
    x2 = _comb(p2)
    p3 = _hop(x2, wp, srcp, dstp)
    rst = _final(h_pad, x1, x2, p3, hl, hr)
    return rst[:N].reshape(N, 1, F)


# speedup vs baseline: 1.0218x; 1.0218x over previous
import functools

import numpy as np
import jax
import jax.numpy as jnp
from jax import lax
from jax.experimental import pallas as pl
from jax.experimental.pallas import tpu as pltpu
from jax.experimental.pallas import tpu_sc as plsc

N = 10000
E = 320000
F = 128
NEG = 0.2
NP = 10240
NC, NS, L = 2, 16, 16
NW = NC * NS
CH = 96
GCH = 108
EW = GCH * CH
EP = NW * EW
TRI = GCH // 3
C10 = float(np.log(1e-10))
RPT = NP // NS

_mesh = plsc.VectorSubcoreMesh(core_axis_name="c", subcore_axis_name="s")
_sc_params = pltpu.CompilerParams(needs_layout_passes=False)


def _proj_body(feat_ref, w_ref, al_ref, ar_ref, h_ref, sc_ref):
    x = feat_ref[...]
    w = w_ref[...]
    h = lax.dot_general(x, w, (((1,), (1,)), ((), ())),
                        preferred_element_type=jnp.float32)
    h_ref[...] = h
    el = lax.dot_general(al_ref[...], h, (((1,), (1,)), ((), ())),
                         preferred_element_type=jnp.float32)
    er = lax.dot_general(ar_ref[...], h, (((1,), (1,)), ((), ())),
                         preferred_element_type=jnp.float32)
    sc_ref[...] = jnp.concatenate([el, er], axis=0)


_BA = 1024
_proj = pl.pallas_call(
    _proj_body,
    grid=(NP // _BA,),
    in_specs=[pl.BlockSpec((_BA, 128), lambda i: (i, 0)),
              pl.BlockSpec((128, 128), lambda i: (0, 0)),
              pl.BlockSpec((1, 128), lambda i: (0, 0)),
              pl.BlockSpec((1, 128), lambda i: (0, 0))],
    out_specs=[pl.BlockSpec((_BA, 128), lambda i: (i, 0)),
               pl.BlockSpec((2, _BA), lambda i: (0, i))],
    out_shape=[jax.ShapeDtypeStruct((NP, 128), jnp.float32),
               jax.ShapeDtypeStruct((2, NP), jnp.float32)],
)


@functools.partial(
    pl.kernel,
    out_type=jax.ShapeDtypeStruct((NW, 4, NP), jnp.float32),
    mesh=_mesh,
    scratch_types=[
        pltpu.VMEM((2, NP), jnp.float32),
        pltpu.VMEM((NP,), jnp.float32),
        pltpu.VMEM((NP,), jnp.float32),
        pltpu.VMEM((NP,), jnp.float32),
        pltpu.VMEM((NP,), jnp.float32),
        pltpu.VMEM((EW,), jnp.int32),
        pltpu.VMEM((EW,), jnp.int32),
    ],
    compiler_params=_sc_params,
)
def _edge_pass1(sc_hbm, src_hbm, dst_hbm, out_hbm,
                nv, a_ss, a_sd, a_do, a_di, se, de):
    cid = lax.axis_index("c")
    sid = lax.axis_index("s")
    wid = cid * NS + sid
    pltpu.sync_copy(sc_hbm, nv)
    pltpu.sync_copy(src_hbm.at[pl.ds(wid * EW, EW)], se)
    pltpu.sync_copy(dst_hbm.at[pl.ds(wid * EW, EW)], de)
    zf = jnp.zeros((16,), jnp.float32)

    def zbody(i, _):
        a_ss[pl.ds(i * 16, 16)] = zf
        a_sd[pl.ds(i * 16, 16)] = zf
        a_do[pl.ds(i * 16, 16)] = zf
        a_di[pl.ds(i * 16, 16)] = zf
        return 0

    lax.fori_loop(0, NP // 16, zbody, 0)
    c0 = jnp.zeros((16,), jnp.int32)
    c1 = jnp.ones((16,), jnp.int32)
    onef = jnp.ones((16,), jnp.float32)

    def gbody(g, _):
        base = g * CH
        for j in range(CH // 16):
            s = se[pl.ds(base + j * 16, 16)]
            d = de[pl.ds(base + j * 16, 16)]
            elv = plsc.load_gather(nv, [c0, s])
            erv = plsc.load_gather(nv, [c1, d])
            t = elv + erv
            ex = jnp.exp(jnp.where(t >= 0.0, t, t * NEG))
            plsc.addupdate_scatter(a_ss, [s], ex)
            plsc.addupdate_scatter(a_sd, [d], ex)
            plsc.addupdate_scatter(a_do, [s], onef)
            plsc.addupdate_scatter(a_di, [d], onef)
        return 0

    lax.fori_loop(0, GCH, gbody, 0)
    pltpu.sync_copy(a_ss, out_hbm.at[wid, 0])
    pltpu.sync_copy(a_sd, out_hbm.at[wid, 1])
    pltpu.sync_copy(a_do, out_hbm.at[wid, 2])
    pltpu.sync_copy(a_di, out_hbm.at[wid, 3])


def _nodecalc_body(p_ref, sc_ref, sig_ref, o_ref):
    p = p_ref[...]
    s = jnp.sum(p, axis=0)
    el = sc_ref[0, :]
    er = sc_ref[1, :]
    ls_src = jnp.log(jnp.maximum(s[0], 1e-38))
    ls_dst = jnp.log(jnp.maximum(s[1], 1e-38))
    lo = -0.5 * jnp.log(jnp.maximum(s[2], 1.0))
    li = 0.5 * jnp.log(jnp.maximum(s[3], 1.0))
    sg = 1.0 / (1.0 + jnp.exp(-sig_ref[0]))
    sgr = jnp.full_like(el, sg)
    o_ref[...] = jnp.stack(
        [el, er, ls_src, ls_dst, lo, li, sgr, jnp.zeros_like(el)], axis=0)


_BC = 2048
_nodecalc = pl.pallas_call(
    _nodecalc_body,
    grid=(NP // _BC,),
    in_specs=[pl.BlockSpec((NW, 4, _BC), lambda i: (0, 0, i)),
              pl.BlockSpec((2, _BC), lambda i: (0, i)),
              pl.BlockSpec(memory_space=pltpu.SMEM)],
    out_specs=pl.BlockSpec((8, _BC), lambda i: (0, i)),
    out_shape=jax.ShapeDtypeStruct((8, NP), jnp.float32),
)


@functools.partial(
    pl.kernel,
    out_type=jax.ShapeDtypeStruct((EP,), jnp.float32),
    mesh=_mesh,
    scratch_types=[
        pltpu.VMEM((8, NP), jnp.float32),
        pltpu.VMEM((EW,), jnp.int32),
        pltpu.VMEM((EW,), jnp.int32),
        pltpu.VMEM((EW,), jnp.float32),
    ],
    compiler_params=_sc_params,
)
def _edge_pass2(nsc_hbm, src_hbm, dst_hbm, out_hbm, nv, se, de, wl):
    cid = lax.axis_index("c")
    sid = lax.axis_index("s")
    wid = cid * NS + sid
    pltpu.sync_copy(nsc_hbm, nv)
    pltpu.sync_copy(src_hbm.at[pl.ds(wid * EW, EW)], se)
    pltpu.sync_copy(dst_hbm.at[pl.ds(wid * EW, EW)], de)
    cs = [jnp.full((16,), k, jnp.int32) for k in range(6)]
    sgv = nv[6, pl.ds(0, 16)]

    @plsc.parallel_loop(0, EW // 16, unroll=2)
    def gbody(v):
        o = v * 16
        s = se[pl.ds(o, 16)]
        d = de[pl.ds(o, 16)]
        elv = plsc.load_gather(nv, [cs[0], s])
        erv = plsc.load_gather(nv, [cs[1], d])
        lss = plsc.load_gather(nv, [cs[2], s])
        lsd = plsc.load_gather(nv, [cs[3], d])
        lov = plsc.load_gather(nv, [cs[4], s])
        liv = plsc.load_gather(nv, [cs[5], d])
        t = elv + erv
        e = jnp.where(t >= 0.0, t, t * NEG)
        las = jnp.maximum(e - lss, C10)
        lad = jnp.maximum(e - lsd, C10)
        wl[pl.ds(o, 16)] = jnp.exp(
            sgv * lad + (1.0 - sgv) * las + lov + liv)
    pltpu.sync_copy(wl, out_hbm.at[pl.ds(wid * EW, EW)])


@functools.partial(
    pl.kernel,
    out_type=jax.ShapeDtypeStruct((NC, NP, 128), jnp.float32),
    mesh=_mesh,
    scratch_types=[
        pltpu.VMEM((EW,), jnp.int32),
        pltpu.VMEM((CH,), jnp.int32),
        pltpu.VMEM((CH,), jnp.int32),
        pltpu.VMEM((CH,), jnp.int32),
        pltpu.VMEM((CH,), jnp.float32),
        pltpu.VMEM((CH,), jnp.float32),
        pltpu.VMEM((CH,), jnp.float32),
        pltpu.VMEM((CH, 128), jnp.float32),
        pltpu.VMEM((CH, 128), jnp.float32),
        pltpu.VMEM((CH, 128), jnp.float32),
        pltpu.VMEM_SHARED((NP, 128), jnp.float32),
        pltpu.SemaphoreType.DMA,
        pltpu.SemaphoreType.DMA,
        pltpu.SemaphoreType.DMA,
        pltpu.SemaphoreType.DMA,
        pltpu.SemaphoreType.DMA,
        pltpu.SemaphoreType.DMA,
        pltpu.SemaphoreType.DMA,
        pltpu.SemaphoreType.DMA,
        pltpu.SemaphoreType.DMA,
    ],
    compiler_params=_sc_params,
)
def _hop(x_hbm, w_hbm, src_hbm, dst_hbm, out_hbm,
         se, db0, db1, db2, wb0, wb1, wb2, rows0, rows1, rows2, acc,
         semg0, semg1, semg2, semi0, semi1, semi2, sems0, sems1, sems2):
    cid = lax.axis_index("c")
    sid = lax.axis_index("s")
    wid = cid * NS + sid
    pltpu.sync_copy(src_hbm.at[pl.ds(wid * EW, EW)], se)
    zf = jnp.zeros((16,), jnp.float32)

    def zrow(r, _):
        for j in range(8):
            rows0[r, pl.ds(j * 16, 16)] = zf
        return 0

    lax.fori_loop(0, CH, zrow, 0)
    base0 = sid * RPT
    for b in range(RPT // CH):
        pltpu.async_copy(rows0, acc.at[pl.ds(base0 + b * CH, CH)], semi0)
    rem = RPT - (RPT // CH) * CH
    if rem:
        pltpu.async_copy(rows0.at[pl.ds(0, rem)],
                         acc.at[pl.ds(base0 + (RPT // CH) * CH, rem)], semi0)
    for b in range(RPT // CH):
        pltpu.make_async_copy(rows0, acc.at[pl.ds(base0 + b * CH, CH)],
                              semi0).wait()
    if rem:
        pltpu.make_async_copy(rows0.at[pl.ds(0, rem)],
                              acc.at[pl.ds(base0 + (RPT // CH) * CH, rem)],
                              semi0).wait()
    plsc.subcore_barrier()

    rows = (rows0, rows1, rows2)
    dbs = (db0, db1, db2)
    wbs = (wb0, wb1, wb2)
    semg = (semg0, semg1, semg2)
    semi = (semi0, semi1, semi2)
    sems = (sems0, sems1, sems2)

    def prefetch(g, slot):
        pltpu.async_copy(w_hbm.at[pl.ds(wid * EW + g * CH, CH)], wbs[slot],
                         semi[slot])
        pltpu.async_copy(dst_hbm.at[pl.ds(wid * EW + g * CH, CH)], dbs[slot],
                         semi[slot])
        pltpu.async_copy(x_hbm.at[se.at[pl.ds(g * CH, CH)]], rows[slot],
                         semg[slot])

    def wait_scatter(slot):
        pltpu.make_async_copy(rows[slot], acc.at[dbs[slot]],
                              sems[slot]).wait()

    def process(g, slot):
        rb = rows[slot]
        wb = wbs[slot]
        pltpu.make_async_copy(w_hbm.at[pl.ds(wid * EW + g * CH, CH)], wb,
                              semi[slot]).wait()
        pltpu.make_async_copy(dst_hbm.at[pl.ds(wid * EW + g * CH, CH)],
                              dbs[slot], semi[slot]).wait()
        pltpu.make_async_copy(x_hbm.at[se.at[pl.ds(g * CH, CH)]], rb,
                              semg[slot]).wait()

        @plsc.parallel_loop(0, CH // 16, unroll=2)
        def sgrp(q):
            wv16 = wb[pl.ds(q * 16, 16)]
            base = q * 16
            for r in range(16):
                wvr = jnp.broadcast_to(wv16[r], (16,))
                for j in range(8):
                    rb[base + r, pl.ds(j * 16, 16)] = (
                        rb[base + r, pl.ds(j * 16, 16)] * wvr)
        pltpu.async_copy(rb, acc.at[dbs[slot]], sems[slot], add=True)

    prefetch(0, 0)
    prefetch(1, 1)

    def rbody(r, _):
        for b in range(3):
            g = 3 * r + b
            slot = b
            slot2 = (b + 2) % 3
            if b == 0:
                @pl.when(r >= 1)
                def _():
                    wait_scatter(slot2)
                prefetch(g + 2, slot2)
            else:
                @pl.when(r < TRI - 1)
                def _():
                    wait_scatter(slot2)
                    prefetch(g + 2, slot2)
            process(g, slot)
        return 0

    lax.fori_loop(0, TRI, rbody, 0)
    for s in range(3):
        wait_scatter(s)
    plsc.subcore_barrier()
    for b in range(RPT // 128):
        r0 = sid * RPT + b * 128
        pltpu.async_copy(acc.at[pl.ds(r0, 128)],
                         out_hbm.at[cid, pl.ds(r0, 128)], semi0)
    for b in range(RPT // 128):
        r0 = sid * RPT + b * 128
        pltpu.make_async_copy(acc.at[pl.ds(r0, 128)],
                              out_hbm.at[cid, pl.ds(r0, 128)], semi0).wait()


def _comb_body(p_ref, o_ref):
    o_ref[...] = p_ref[0] + p_ref[1]


_BB = 1024
_comb = pl.pallas_call(
    _comb_body,
    grid=(NP // _BB,),
    in_specs=[pl.BlockSpec((NC, _BB, 128), lambda i: (0, i, 0))],
    out_specs=pl.BlockSpec((_BB, 128), lambda i: (i, 0)),
    out_shape=jax.ShapeDtypeStruct((NP, 128), jnp.float32),
)


def _final_body(h_ref, x1_ref, x2_ref, p3_ref, hl_ref, hr_ref, o_ref):
    h = h_ref[...]
    x1 = x1_ref[...]
    x2 = x2_ref[...]
    x3 = p3_ref[0] + p3_ref[1]
    hl = hl_ref[...]
    hr = hr_ref[...]
    al = jnp.sum(h * hl, axis=1, keepdims=True)
    xs = (h, x1, x2, x3)
    ls = []
    for x in xs:
        v = al + jnp.sum(x * hr, axis=1, keepdims=True)
        ls.append(jnp.where(v >= 0.0, v, v * NEG))
    m = jnp.maximum(jnp.maximum(ls[0], ls[1]), jnp.maximum(ls[2], ls[3]))
    es = [jnp.exp(v - m) for v in ls]
    tot = es[0] + es[1] + es[2] + es[3]
    o_ref[...] = (h * es[0] + x1 * es[1] + x2 * es[2] + x3 * es[3]) / tot


_final = pl.pallas_call(
    _final_body,
    grid=(NP // _BA,),
    in_specs=[pl.BlockSpec((_BA, 128), lambda i: (i, 0)),
              pl.BlockSpec((_BA, 128), lambda i: (i, 0)),
              pl.BlockSpec((_BA, 128), lambda i: (i, 0)),
              pl.BlockSpec((NC, _BA, 128), lambda i: (0, i, 0)),
              pl.BlockSpec((1, 128), lambda i: (0, 0)),
              pl.BlockSpec((1, 128), lambda i: (0, 0))],
    out_specs=pl.BlockSpec((_BA, 128), lambda i: (i, 0)),
    out_shape=jax.ShapeDtypeStruct((NP, 128), jnp.float32),
)


def kernel(feat, W_fc, attn_l, attn_r, hop_attn_l, hop_attn_r, sigma,
           edge_index):
    feat_p = jnp.pad(feat, ((0, NP - N), (0, 0)))
    al = attn_l.reshape(1, F)
    ar = attn_r.reshape(1, F)
    hl = hop_attn_l.reshape(1, F)
    hr = hop_attn_r.reshape(1, F)
    padn = EP - E
    pad_idx = N + (jnp.arange(padn, dtype=jnp.int32) % (NP - N))
    srcp = jnp.concatenate([edge_index[0], pad_idx])
    dstp = jnp.concatenate([edge_index[1], pad_idx])

    h_pad, sc1 = _proj(feat_p, W_fc, al, ar)
    part1 = _edge_pass1(sc1, srcp, dstp)
    nsc = _nodecalc(part1, sc1, sigma)
    wp = _edge_pass2(nsc, srcp, dstp)
    p1 = _hop(h_pad, wp, srcp, dstp)
    x1 = _comb(p1)
    p2 = _hop(x1, wp, srcp, dstp)
    x2 = _comb(p2)
    p3 = _hop(x2, wp, srcp, dstp)
    rst = _final(h_pad, x1, x2, p3, hl, hr)
    return rst[:N].reshape(N, 1, F)


# speedup vs baseline: 1.0325x; 1.0104x over previous
import functools

import numpy as np
import jax
import jax.numpy as jnp
from jax import lax
from jax.experimental import pallas as pl
from jax.experimental.pallas import tpu as pltpu
from jax.experimental.pallas import tpu_sc as plsc

N = 10000
E = 320000
F = 128
NEG = 0.2
NP = 10240
NC, NS, L = 2, 16, 16
NW = NC * NS
CH = 96
GCH = 108
EW = GCH * CH
EP = NW * EW
TRI = GCH // 3
C10 = float(np.log(1e-10))
RPT = NP // NS

_mesh = plsc.VectorSubcoreMesh(core_axis_name="c", subcore_axis_name="s")
_sc_params = pltpu.CompilerParams(needs_layout_passes=False)


def _proj_body(feat_ref, w_ref, al_ref, ar_ref, h_ref, sc_ref):
    x = feat_ref[...]
    w = w_ref[...]
    h = lax.dot_general(x, w, (((1,), (1,)), ((), ())),
                        preferred_element_type=jnp.float32)
    h_ref[...] = h
    el = lax.dot_general(al_ref[...], h, (((1,), (1,)), ((), ())),
                         preferred_element_type=jnp.float32)
    er = lax.dot_general(ar_ref[...], h, (((1,), (1,)), ((), ())),
                         preferred_element_type=jnp.float32)
    sc_ref[...] = jnp.concatenate([el, er], axis=0)


_BA = 1024
_proj = pl.pallas_call(
    _proj_body,
    grid=(NP // _BA,),
    in_specs=[pl.BlockSpec((_BA, 128), lambda i: (i, 0)),
              pl.BlockSpec((128, 128), lambda i: (0, 0)),
              pl.BlockSpec((1, 128), lambda i: (0, 0)),
              pl.BlockSpec((1, 128), lambda i: (0, 0))],
    out_specs=[pl.BlockSpec((_BA, 128), lambda i: (i, 0)),
               pl.BlockSpec((2, _BA), lambda i: (0, i))],
    out_shape=[jax.ShapeDtypeStruct((NP, 128), jnp.float32),
               jax.ShapeDtypeStruct((2, NP), jnp.float32)],
)


@functools.partial(
    pl.kernel,
    out_type=jax.ShapeDtypeStruct((NW, 4, NP), jnp.float32),
    mesh=_mesh,
    scratch_types=[
        pltpu.VMEM((2, NP), jnp.float32),
        pltpu.VMEM((NP,), jnp.float32),
        pltpu.VMEM((NP,), jnp.float32),
        pltpu.VMEM((NP,), jnp.float32),
        pltpu.VMEM((NP,), jnp.float32),
        pltpu.VMEM((EW,), jnp.int32),
        pltpu.VMEM((EW,), jnp.int32),
    ],
    compiler_params=_sc_params,
)
def _edge_pass1(sc_hbm, src_hbm, dst_hbm, out_hbm,
                nv, a_ss, a_sd, a_do, a_di, se, de):
    cid = lax.axis_index("c")
    sid = lax.axis_index("s")
    wid = cid * NS + sid
    pltpu.sync_copy(sc_hbm, nv)
    pltpu.sync_copy(src_hbm.at[pl.ds(wid * EW, EW)], se)
    pltpu.sync_copy(dst_hbm.at[pl.ds(wid * EW, EW)], de)
    zf = jnp.zeros((16,), jnp.float32)

    def zbody(i, _):
        a_ss[pl.ds(i * 16, 16)] = zf
        a_sd[pl.ds(i * 16, 16)] = zf
        a_do[pl.ds(i * 16, 16)] = zf
        a_di[pl.ds(i * 16, 16)] = zf
        return 0

    lax.fori_loop(0, NP // 16, zbody, 0)
    c0 = jnp.zeros((16,), jnp.int32)
    c1 = jnp.ones((16,), jnp.int32)
    onef = jnp.ones((16,), jnp.float32)

    def gbody(g, _):
        base = g * CH
        for j in range(CH // 16):
            s = se[pl.ds(base + j * 16, 16)]
            d = de[pl.ds(base + j * 16, 16)]
            elv = plsc.load_gather(nv, [c0, s])
            erv = plsc.load_gather(nv, [c1, d])
            t = elv + erv
            ex = jnp.exp(jnp.where(t >= 0.0, t, t * NEG))
            plsc.addupdate_scatter(a_ss, [s], ex)
            plsc.addupdate_scatter(a_sd, [d], ex)
            plsc.addupdate_scatter(a_do, [s], onef)
            plsc.addupdate_scatter(a_di, [d], onef)
        return 0

    lax.fori_loop(0, GCH, gbody, 0)
    pltpu.sync_copy(a_ss, out_hbm.at[wid, 0])
    pltpu.sync_copy(a_sd, out_hbm.at[wid, 1])
    pltpu.sync_copy(a_do, out_hbm.at[wid, 2])
    pltpu.sync_copy(a_di, out_hbm.at[wid, 3])


def _nodecalc_body(p_ref, sc_ref, sig_ref, o_ref):
    p = p_ref[...]
    s = jnp.sum(p, axis=0)
    el = sc_ref[0, :]
    er = sc_ref[1, :]
    ls_src = jnp.log(jnp.maximum(s[0], 1e-38))
    ls_dst = jnp.log(jnp.maximum(s[1], 1e-38))
    lo = -0.5 * jnp.log(jnp.maximum(s[2], 1.0))
    li = 0.5 * jnp.log(jnp.maximum(s[3], 1.0))
    sg = 1.0 / (1.0 + jnp.exp(-sig_ref[0]))
    sgr = jnp.full_like(el, sg)
    o_ref[...] = jnp.stack(
        [el, er, ls_src, ls_dst, lo, li, sgr, jnp.zeros_like(el)], axis=0)


_BC = 2048
_nodecalc = pl.pallas_call(
    _nodecalc_body,
    grid=(NP // _BC,),
    in_specs=[pl.BlockSpec((NW, 4, _BC), lambda i: (0, 0, i)),
              pl.BlockSpec((2, _BC), lambda i: (0, i)),
              pl.BlockSpec(memory_space=pltpu.SMEM)],
    out_specs=pl.BlockSpec((8, _BC), lambda i: (0, i)),
    out_shape=jax.ShapeDtypeStruct((8, NP), jnp.float32),
)


@functools.partial(
    pl.kernel,
    out_type=jax.ShapeDtypeStruct((EP,), jnp.float32),
    mesh=_mesh,
    scratch_types=[
        pltpu.VMEM((8, NP), jnp.float32),
        pltpu.VMEM((EW,), jnp.int32),
        pltpu.VMEM((EW,), jnp.int32),
        pltpu.VMEM((EW,), jnp.float32),
    ],
    compiler_params=_sc_params,
)
def _edge_pass2(nsc_hbm, src_hbm, dst_hbm, out_hbm, nv, se, de, wl):
    cid = lax.axis_index("c")
    sid = lax.axis_index("s")
    wid = cid * NS + sid
    pltpu.sync_copy(nsc_hbm, nv)
    pltpu.sync_copy(src_hbm.at[pl.ds(wid * EW, EW)], se)
    pltpu.sync_copy(dst_hbm.at[pl.ds(wid * EW, EW)], de)
    cs = [jnp.full((16,), k, jnp.int32) for k in range(6)]
    sgv = nv[6, pl.ds(0, 16)]

    @plsc.parallel_loop(0, EW // 16, unroll=2)
    def gbody(v):
        o = v * 16
        s = se[pl.ds(o, 16)]
        d = de[pl.ds(o, 16)]
        elv = plsc.load_gather(nv, [cs[0], s])
        erv = plsc.load_gather(nv, [cs[1], d])
        lss = plsc.load_gather(nv, [cs[2], s])
        lsd = plsc.load_gather(nv, [cs[3], d])
        lov = plsc.load_gather(nv, [cs[4], s])
        liv = plsc.load_gather(nv, [cs[5], d])
        t = elv + erv
        e = jnp.where(t >= 0.0, t, t * NEG)
        las = jnp.maximum(e - lss, C10)
        lad = jnp.maximum(e - lsd, C10)
        wl[pl.ds(o, 16)] = jnp.exp(
            sgv * lad + (1.0 - sgv) * las + lov + liv)
    pltpu.sync_copy(wl, out_hbm.at[pl.ds(wid * EW, EW)])


@functools.partial(
    pl.kernel,
    out_type=jax.ShapeDtypeStruct((NC, NP, 128), jnp.float32),
    mesh=_mesh,
    scratch_types=[
        pltpu.VMEM((EW,), jnp.int32),
        pltpu.VMEM((CH,), jnp.int32),
        pltpu.VMEM((CH,), jnp.int32),
        pltpu.VMEM((CH,), jnp.int32),
        pltpu.VMEM((CH,), jnp.float32),
        pltpu.VMEM((CH,), jnp.float32),
        pltpu.VMEM((CH,), jnp.float32),
        pltpu.VMEM((CH, 128), jnp.float32),
        pltpu.VMEM((CH, 128), jnp.float32),
        pltpu.VMEM((CH, 128), jnp.float32),
        pltpu.VMEM_SHARED((NP, 128), jnp.float32),
        pltpu.SemaphoreType.DMA,
        pltpu.SemaphoreType.DMA,
        pltpu.SemaphoreType.DMA,
        pltpu.SemaphoreType.DMA,
        pltpu.SemaphoreType.DMA,
        pltpu.SemaphoreType.DMA,
        pltpu.SemaphoreType.DMA,
        pltpu.SemaphoreType.DMA,
        pltpu.SemaphoreType.DMA,
    ],
    compiler_params=_sc_params,
)
def _hop(x_hbm, w_hbm, src_hbm, dst_hbm, out_hbm,
         se, db0, db1, db2, wb0, wb1, wb2, rows0, rows1, rows2, acc,
         semg0, semg1, semg2, semi0, semi1, semi2, sems0, sems1, sems2):
    cid = lax.axis_index("c")
    sid = lax.axis_index("s")
    wid = cid * NS + sid
    pltpu.sync_copy(src_hbm.at[pl.ds(wid * EW, EW)], se)
    zf = jnp.zeros((16,), jnp.float32)

    def zrow(r, _):
        for j in range(8):
            rows0[r, pl.ds(j * 16, 16)] = zf
        return 0

    lax.fori_loop(0, CH, zrow, 0)
    base0 = sid * RPT
    for b in range(RPT // CH):
        pltpu.async_copy(rows0, acc.at[pl.ds(base0 + b * CH, CH)], semi0)
    rem = RPT - (RPT // CH) * CH
    if rem:
        pltpu.async_copy(rows0.at[pl.ds(0, rem)],
                         acc.at[pl.ds(base0 + (RPT // CH) * CH, rem)], semi0)
    for b in range(RPT // CH):
        pltpu.make_async_copy(rows0, acc.at[pl.ds(base0 + b * CH, CH)],
                              semi0).wait()
    if rem:
        pltpu.make_async_copy(rows0.at[pl.ds(0, rem)],
                              acc.at[pl.ds(base0 + (RPT // CH) * CH, rem)],
                              semi0).wait()
    plsc.subcore_barrier()

    rows = (rows0, rows1, rows2)
    dbs = (db0, db1, db2)
    wbs = (wb0, wb1, wb2)
    semg = (semg0, semg1, semg2)
    semi = (semi0, semi1, semi2)
    sems = (sems0, sems1, sems2)

    def prefetch(g, slot):
        pltpu.async_copy(w_hbm.at[pl.ds(wid * EW + g * CH, CH)], wbs[slot],
                         semi[slot])
        pltpu.async_copy(dst_hbm.at[pl.ds(wid * EW + g * CH, CH)], dbs[slot],
                         semi[slot])
        pltpu.async_copy(x_hbm.at[se.at[pl.ds(g * CH, CH)]], rows[slot],
                         semg[slot])

    def wait_scatter(slot):
        pltpu.make_async_copy(rows[slot], acc.at[dbs[slot]],
                              sems[slot]).wait()

    def process(g, slot):
        rb = rows[slot]
        wb = wbs[slot]
        pltpu.make_async_copy(w_hbm.at[pl.ds(wid * EW + g * CH, CH)], wb,
                              semi[slot]).wait()
        pltpu.make_async_copy(dst_hbm.at[pl.ds(wid * EW + g * CH, CH)],
                              dbs[slot], semi[slot]).wait()
        pltpu.make_async_copy(x_hbm.at[se.at[pl.ds(g * CH, CH)]], rb,
                              semg[slot]).wait()

        @plsc.parallel_loop(0, CH // 16, unroll=2)
        def sgrp(q):
            wv16 = wb[pl.ds(q * 16, 16)]
            base = q * 16
            for r in range(16):
                wvr = jnp.broadcast_to(wv16[r], (16,))
                for j in range(8):
                    rb[base + r, pl.ds(j * 16, 16)] = (
                        rb[base + r, pl.ds(j * 16, 16)] * wvr)
        pltpu.async_copy(rb, acc.at[dbs[slot]], sems[slot], add=True)

    prefetch(0, 0)
    prefetch(1, 1)

    def rbody(r, _):
        for b in range(3):
            g = 3 * r + b
            slot = b
            slot2 = (b + 2) % 3
            process(g, slot)
            if b == 0:
                @pl.when(r >= 1)
                def _():
                    wait_scatter(slot2)
                prefetch(g + 2, slot2)
            else:
                @pl.when(r < TRI - 1)
                def _():
                    wait_scatter(slot2)
                    prefetch(g + 2, slot2)
        return 0

    lax.fori_loop(0, TRI, rbody, 0)
    for s in range(3):
        wait_scatter(s)
    plsc.subcore_barrier()
    for b in range(RPT // 128):
        r0 = sid * RPT + b * 128
        pltpu.async_copy(acc.at[pl.ds(r0, 128)],
                         out_hbm.at[cid, pl.ds(r0, 128)], semi0)
    for b in range(RPT // 128):
        r0 = sid * RPT + b * 128
        pltpu.make_async_copy(acc.at[pl.ds(r0, 128)],
                              out_hbm.at[cid, pl.ds(r0, 128)], semi0).wait()


def _comb_body(p_ref, o_ref):
    o_ref[...] = p_ref[0] + p_ref[1]


_BB = 1024
_comb = pl.pallas_call(
    _comb_body,
    grid=(NP // _BB,),
    in_specs=[pl.BlockSpec((NC, _BB, 128), lambda i: (0, i, 0))],
    out_specs=pl.BlockSpec((_BB, 128), lambda i: (i, 0)),
    out_shape=jax.ShapeDtypeStruct((NP, 128), jnp.float32),
)


def _final_body(h_ref, x1_ref, x2_ref, p3_ref, hl_ref, hr_ref, o_ref):
    h = h_ref[...]
    x1 = x1_ref[...]
    x2 = x2_ref[...]
    x3 = p3_ref[0] + p3_ref[1]
    hl = hl_ref[...]
    hr = hr_ref[...]
    al = jnp.sum(h * hl, axis=1, keepdims=True)
    xs = (h, x1, x2, x3)
    ls = []
    for x in xs:
        v = al + jnp.sum(x * hr, axis=1, keepdims=True)
        ls.append(jnp.where(v >= 0.0, v, v * NEG))
    m = jnp.maximum(jnp.maximum(ls[0], ls[1]), jnp.maximum(ls[2], ls[3]))
    es = [jnp.exp(v - m) for v in ls]
    tot = es[0] + es[1] + es[2] + es[3]
    o_ref[...] = (h * es[0] + x1 * es[1] + x2 * es[2] + x3 * es[3]) / tot


_final = pl.pallas_call(
    _final_body,
    grid=(NP // _BA,),
    in_specs=[pl.BlockSpec((_BA, 128), lambda i: (i, 0)),
              pl.BlockSpec((_BA, 128), lambda i: (i, 0)),
              pl.BlockSpec((_BA, 128), lambda i: (i, 0)),
              pl.BlockSpec((NC, _BA, 128), lambda i: (0, i, 0)),
              pl.BlockSpec((1, 128), lambda i: (0, 0)),
              pl.BlockSpec((1, 128), lambda i: (0, 0))],
    out_specs=pl.BlockSpec((_BA, 128), lambda i: (i, 0)),
    out_shape=jax.ShapeDtypeStruct((NP, 128), jnp.float32),
)


def kernel(feat, W_fc, attn_l, attn_r, hop_attn_l, hop_attn_r, sigma,
           edge_index):
    feat_p = jnp.pad(feat, ((0, NP - N), (0, 0)))
    al = attn_l.reshape(1, F)
    ar = attn_r.reshape(1, F)
    hl = hop_attn_l.reshape(1, F)
    hr = hop_attn_r.reshape(1, F)
    padn = EP - E
    pad_idx = N + (jnp.arange(padn, dtype=jnp.int32) % (NP - N))
    srcp = jnp.concatenate([edge_index[0], pad_idx])
    dstp = jnp.concatenate([edge_index[1], pad_idx])

    h_pad, sc1 = _proj(feat_p, W_fc, al, ar)
    part1 = _edge_pass1(sc1, srcp, dstp)
    nsc = _nodecalc(part1, sc1, sigma)
    wp = _edge_pass2(nsc, srcp, dstp)
    p1 = _hop(h_pad, wp, srcp, dstp)
    x1 = _comb(p1)
    p2 = _hop(x1, wp, srcp, dstp)
    x2 = _comb(p2)
    p3 = _hop(x2, wp, srcp, dstp)
    rst = _final(h_pad, x1, x2, p3, hl, hr)
    return rst[:N].reshape(N, 1, F)


# speedup vs baseline: 1.0678x; 1.0342x over previous
import functools

import numpy as np
import jax
import jax.numpy as jnp
from jax import lax
from jax.experimental import pallas as pl
from jax.experimental.pallas import tpu as pltpu
from jax.experimental.pallas import tpu_sc as plsc

N = 10000
E = 320000
F = 128
NEG = 0.2
NP = 10240
NC, NS, L = 2, 16, 16
NW = NC * NS
CH = 64
GCH = 160
EW = GCH * CH
EP = NW * EW
QUAD = GCH // 4
C10 = float(np.log(1e-10))
RPT = NP // NS

_mesh = plsc.VectorSubcoreMesh(core_axis_name="c", subcore_axis_name="s")
_sc_params = pltpu.CompilerParams(needs_layout_passes=False)


def _proj_body(feat_ref, w_ref, al_ref, ar_ref, h_ref, sc_ref):
    x = feat_ref[...]
    w = w_ref[...]
    h = lax.dot_general(x, w, (((1,), (1,)), ((), ())),
                        preferred_element_type=jnp.float32)
    h_ref[...] = h
    el = lax.dot_general(al_ref[...], h, (((1,), (1,)), ((), ())),
                         preferred_element_type=jnp.float32)
    er = lax.dot_general(ar_ref[...], h, (((1,), (1,)), ((), ())),
                         preferred_element_type=jnp.float32)
    sc_ref[...] = jnp.concatenate([el, er], axis=0)


_BA = 1024
_proj = pl.pallas_call(
    _proj_body,
    grid=(NP // _BA,),
    in_specs=[pl.BlockSpec((_BA, 128), lambda i: (i, 0)),
              pl.BlockSpec((128, 128), lambda i: (0, 0)),
              pl.BlockSpec((1, 128), lambda i: (0, 0)),
              pl.BlockSpec((1, 128), lambda i: (0, 0))],
    out_specs=[pl.BlockSpec((_BA, 128), lambda i: (i, 0)),
               pl.BlockSpec((2, _BA), lambda i: (0, i))],
    out_shape=[jax.ShapeDtypeStruct((NP, 128), jnp.float32),
               jax.ShapeDtypeStruct((2, NP), jnp.float32)],
)


@functools.partial(
    pl.kernel,
    out_type=jax.ShapeDtypeStruct((NW, 4, NP), jnp.float32),
    mesh=_mesh,
    scratch_types=[
        pltpu.VMEM((2, NP), jnp.float32),
        pltpu.VMEM((NP,), jnp.float32),
        pltpu.VMEM((NP,), jnp.float32),
        pltpu.VMEM((NP,), jnp.float32),
        pltpu.VMEM((NP,), jnp.float32),
        pltpu.VMEM((EW,), jnp.int32),
        pltpu.VMEM((EW,), jnp.int32),
    ],
    compiler_params=_sc_params,
)
def _edge_pass1(sc_hbm, src_hbm, dst_hbm, out_hbm,
                nv, a_ss, a_sd, a_do, a_di, se, de):
    cid = lax.axis_index("c")
    sid = lax.axis_index("s")
    wid = cid * NS + sid
    pltpu.sync_copy(sc_hbm, nv)
    pltpu.sync_copy(src_hbm.at[pl.ds(wid * EW, EW)], se)
    pltpu.sync_copy(dst_hbm.at[pl.ds(wid * EW, EW)], de)
    zf = jnp.zeros((16,), jnp.float32)

    def zbody(i, _):
        a_ss[pl.ds(i * 16, 16)] = zf
        a_sd[pl.ds(i * 16, 16)] = zf
        a_do[pl.ds(i * 16, 16)] = zf
        a_di[pl.ds(i * 16, 16)] = zf
        return 0

    lax.fori_loop(0, NP // 16, zbody, 0)
    c0 = jnp.zeros((16,), jnp.int32)
    c1 = jnp.ones((16,), jnp.int32)
    onef = jnp.ones((16,), jnp.float32)

    def gbody(g, _):
        base = g * CH
        for j in range(CH // 16):
            s = se[pl.ds(base + j * 16, 16)]
            d = de[pl.ds(base + j * 16, 16)]
            elv = plsc.load_gather(nv, [c0, s])
            erv = plsc.load_gather(nv, [c1, d])
            t = elv + erv
            ex = jnp.exp(jnp.where(t >= 0.0, t, t * NEG))
            plsc.addupdate_scatter(a_ss, [s], ex)
            plsc.addupdate_scatter(a_sd, [d], ex)
            plsc.addupdate_scatter(a_do, [s], onef)
            plsc.addupdate_scatter(a_di, [d], onef)
        return 0

    lax.fori_loop(0, GCH, gbody, 0)
    pltpu.sync_copy(a_ss, out_hbm.at[wid, 0])
    pltpu.sync_copy(a_sd, out_hbm.at[wid, 1])
    pltpu.sync_copy(a_do, out_hbm.at[wid, 2])
    pltpu.sync_copy(a_di, out_hbm.at[wid, 3])


def _nodecalc_body(p_ref, sc_ref, sig_ref, o_ref):
    p = p_ref[...]
    s = jnp.sum(p, axis=0)
    el = sc_ref[0, :]
    er = sc_ref[1, :]
    ls_src = jnp.log(jnp.maximum(s[0], 1e-38))
    ls_dst = jnp.log(jnp.maximum(s[1], 1e-38))
    lo = -0.5 * jnp.log(jnp.maximum(s[2], 1.0))
    li = 0.5 * jnp.log(jnp.maximum(s[3], 1.0))
    sg = 1.0 / (1.0 + jnp.exp(-sig_ref[0]))
    sgr = jnp.full_like(el, sg)
    o_ref[...] = jnp.stack(
        [el, er, ls_src, ls_dst, lo, li, sgr, jnp.zeros_like(el)], axis=0)


_BC = 2048
_nodecalc = pl.pallas_call(
    _nodecalc_body,
    grid=(NP // _BC,),
    in_specs=[pl.BlockSpec((NW, 4, _BC), lambda i: (0, 0, i)),
              pl.BlockSpec((2, _BC), lambda i: (0, i)),
              pl.BlockSpec(memory_space=pltpu.SMEM)],
    out_specs=pl.BlockSpec((8, _BC), lambda i: (0, i)),
    out_shape=jax.ShapeDtypeStruct((8, NP), jnp.float32),
)


@functools.partial(
    pl.kernel,
    out_type=jax.ShapeDtypeStruct((EP,), jnp.float32),
    mesh=_mesh,
    scratch_types=[
        pltpu.VMEM((8, NP), jnp.float32),
        pltpu.VMEM((EW,), jnp.int32),
        pltpu.VMEM((EW,), jnp.int32),
        pltpu.VMEM((EW,), jnp.float32),
    ],
    compiler_params=_sc_params,
)
def _edge_pass2(nsc_hbm, src_hbm, dst_hbm, out_hbm, nv, se, de, wl):
    cid = lax.axis_index("c")
    sid = lax.axis_index("s")
    wid = cid * NS + sid
    pltpu.sync_copy(nsc_hbm, nv)
    pltpu.sync_copy(src_hbm.at[pl.ds(wid * EW, EW)], se)
    pltpu.sync_copy(dst_hbm.at[pl.ds(wid * EW, EW)], de)
    cs = [jnp.full((16,), k, jnp.int32) for k in range(6)]
    sgv = nv[6, pl.ds(0, 16)]

    @plsc.parallel_loop(0, EW // 16, unroll=2)
    def gbody(v):
        o = v * 16
        s = se[pl.ds(o, 16)]
        d = de[pl.ds(o, 16)]
        elv = plsc.load_gather(nv, [cs[0], s])
        erv = plsc.load_gather(nv, [cs[1], d])
        lss = plsc.load_gather(nv, [cs[2], s])
        lsd = plsc.load_gather(nv, [cs[3], d])
        lov = plsc.load_gather(nv, [cs[4], s])
        liv = plsc.load_gather(nv, [cs[5], d])
        t = elv + erv
        e = jnp.where(t >= 0.0, t, t * NEG)
        las = jnp.maximum(e - lss, C10)
        lad = jnp.maximum(e - lsd, C10)
        wl[pl.ds(o, 16)] = jnp.exp(
            sgv * lad + (1.0 - sgv) * las + lov + liv)
    pltpu.sync_copy(wl, out_hbm.at[pl.ds(wid * EW, EW)])


@functools.partial(
    pl.kernel,
    out_type=jax.ShapeDtypeStruct((NC, NP, 128), jnp.float32),
    mesh=_mesh,
    scratch_types=[
        pltpu.VMEM((EW,), jnp.int32),
        pltpu.VMEM((CH,), jnp.int32),
        pltpu.VMEM((CH,), jnp.int32),
        pltpu.VMEM((CH,), jnp.int32),
        pltpu.VMEM((CH,), jnp.int32),
        pltpu.VMEM((CH,), jnp.float32),
        pltpu.VMEM((CH,), jnp.float32),
        pltpu.VMEM((CH,), jnp.float32),
        pltpu.VMEM((CH,), jnp.float32),
        pltpu.VMEM((CH, 128), jnp.float32),
        pltpu.VMEM((CH, 128), jnp.float32),
        pltpu.VMEM((CH, 128), jnp.float32),
        pltpu.VMEM((CH, 128), jnp.float32),
        pltpu.VMEM_SHARED((NP, 128), jnp.float32),
    ] + [pltpu.SemaphoreType.DMA] * 12,
    compiler_params=_sc_params,
)
def _hop(x_hbm, w_hbm, src_hbm, dst_hbm, out_hbm,
         se, db0, db1, db2, db3, wb0, wb1, wb2, wb3,
         rows0, rows1, rows2, rows3, acc,
         semg0, semg1, semg2, semg3, semi0, semi1, semi2, semi3,
         sems0, sems1, sems2, sems3):
    cid = lax.axis_index("c")
    sid = lax.axis_index("s")
    wid = cid * NS + sid
    pltpu.sync_copy(src_hbm.at[pl.ds(wid * EW, EW)], se)
    zf = jnp.zeros((16,), jnp.float32)

    def zrow(r, _):
        for j in range(8):
            rows0[r, pl.ds(j * 16, 16)] = zf
        return 0

    lax.fori_loop(0, CH, zrow, 0)
    base0 = sid * RPT
    for b in range(RPT // CH):
        pltpu.async_copy(rows0, acc.at[pl.ds(base0 + b * CH, CH)], semi0)
    rem = RPT - (RPT // CH) * CH
    if rem:
        pltpu.async_copy(rows0.at[pl.ds(0, rem)],
                         acc.at[pl.ds(base0 + (RPT // CH) * CH, rem)], semi0)
    for b in range(RPT // CH):
        pltpu.make_async_copy(rows0, acc.at[pl.ds(base0 + b * CH, CH)],
                              semi0).wait()
    if rem:
        pltpu.make_async_copy(rows0.at[pl.ds(0, rem)],
                              acc.at[pl.ds(base0 + (RPT // CH) * CH, rem)],
                              semi0).wait()
    plsc.subcore_barrier()

    rows = (rows0, rows1, rows2, rows3)
    dbs = (db0, db1, db2, db3)
    wbs = (wb0, wb1, wb2, wb3)
    semg = (semg0, semg1, semg2, semg3)
    semi = (semi0, semi1, semi2, semi3)
    sems = (sems0, sems1, sems2, sems3)

    def prefetch(g, slot):
        pltpu.async_copy(w_hbm.at[pl.ds(wid * EW + g * CH, CH)], wbs[slot],
                         semi[slot])
        pltpu.async_copy(dst_hbm.at[pl.ds(wid * EW + g * CH, CH)], dbs[slot],
                         semi[slot])
        pltpu.async_copy(x_hbm.at[se.at[pl.ds(g * CH, CH)]], rows[slot],
                         semg[slot])

    def wait_scatter(slot):
        pltpu.make_async_copy(rows[slot], acc.at[dbs[slot]],
                              sems[slot]).wait()

    def process(g, slot):
        rb = rows[slot]
        wb = wbs[slot]
        pltpu.make_async_copy(w_hbm.at[pl.ds(wid * EW + g * CH, CH)], wb,
                              semi[slot]).wait()
        pltpu.make_async_copy(dst_hbm.at[pl.ds(wid * EW + g * CH, CH)],
                              dbs[slot], semi[slot]).wait()
        pltpu.make_async_copy(x_hbm.at[se.at[pl.ds(g * CH, CH)]], rb,
                              semg[slot]).wait()

        @plsc.parallel_loop(0, CH // 16, unroll=2)
        def sgrp(q):
            wv16 = wb[pl.ds(q * 16, 16)]
            base = q * 16
            for r in range(16):
                wvr = jnp.broadcast_to(wv16[r], (16,))
                for j in range(8):
                    rb[base + r, pl.ds(j * 16, 16)] = (
                        rb[base + r, pl.ds(j * 16, 16)] * wvr)
        pltpu.async_copy(rb, acc.at[dbs[slot]], sems[slot], add=True)

    prefetch(0, 0)
    prefetch(1, 1)

    def rbody(r, _):
        for b in range(4):
            g = 4 * r + b
            slot = b
            slotp = (b + 2) % 4
            if b <= 1:
                @pl.when(r >= 1)
                def _():
                    wait_scatter(slotp)
                prefetch(g + 2, slotp)
            else:
                @pl.when(r < QUAD - 1)
                def _():
                    wait_scatter(slotp)
                    prefetch(g + 2, slotp)
            process(g, slot)
        return 0

    lax.fori_loop(0, QUAD, rbody, 0)
    for s in range(4):
        wait_scatter(s)
    plsc.subcore_barrier()
    for b in range(RPT // 128):
        r0 = sid * RPT + b * 128
        pltpu.async_copy(acc.at[pl.ds(r0, 128)],
                         out_hbm.at[cid, pl.ds(r0, 128)], semi0)
    for b in range(RPT // 128):
        r0 = sid * RPT + b * 128
        pltpu.make_async_copy(acc.at[pl.ds(r0, 128)],
                              out_hbm.at[cid, pl.ds(r0, 128)], semi0).wait()


def _comb_body(p_ref, o_ref):
    o_ref[...] = p_ref[0] + p_ref[1]


_BB = 1024
_comb = pl.pallas_call(
    _comb_body,
    grid=(NP // _BB,),
    in_specs=[pl.BlockSpec((NC, _BB, 128), lambda i: (0, i, 0))],
    out_specs=pl.BlockSpec((_BB, 128), lambda i: (i, 0)),
    out_shape=jax.ShapeDtypeStruct((NP, 128), jnp.float32),
)


def _final_body(h_ref, x1_ref, x2_ref, p3_ref, hl_ref, hr_ref, o_ref):
    h = h_ref[...]
    x1 = x1_ref[...]
    x2 = x2_ref[...]
    x3 = p3_ref[0] + p3_ref[1]
    hl = hl_ref[...]
    hr = hr_ref[...]
    al = jnp.sum(h * hl, axis=1, keepdims=True)
    xs = (h, x1, x2, x3)
    ls = []
    for x in xs:
        v = al + jnp.sum(x * hr, axis=1, keepdims=True)
        ls.append(jnp.where(v >= 0.0, v, v * NEG))
    m = jnp.maximum(jnp.maximum(ls[0], ls[1]), jnp.maximum(ls[2], ls[3]))
    es = [jnp.exp(v - m) for v in ls]
    tot = es[0] + es[1] + es[2] + es[3]
    o_ref[...] = (h * es[0] + x1 * es[1] + x2 * es[2] + x3 * es[3]) / tot


_final = pl.pallas_call(
    _final_body,
    grid=(NP // _BA,),
    in_specs=[pl.BlockSpec((_BA, 128), lambda i: (i, 0)),
              pl.BlockSpec((_BA, 128), lambda i: (i, 0)),
              pl.BlockSpec((_BA, 128), lambda i: (i, 0)),
              pl.BlockSpec((NC, _BA, 128), lambda i: (0, i, 0)),
              pl.BlockSpec((1, 128), lambda i: (0, 0)),
              pl.BlockSpec((1, 128), lambda i: (0, 0))],
    out_specs=pl.BlockSpec((_BA, 128), lambda i: (i, 0)),
    out_shape=jax.ShapeDtypeStruct((NP, 128), jnp.float32),
)


def kernel(feat, W_fc, attn_l, attn_r, hop_attn_l, hop_attn_r, sigma,
           edge_index):
    feat_p = jnp.pad(feat, ((0, NP - N), (0, 0)))
    al = attn_l.reshape(1, F)
    ar = attn_r.reshape(1, F)
    hl = hop_attn_l.reshape(1, F)
    hr = hop_attn_r.reshape(1, F)
    padn = EP - E
    pad_idx = N + (jnp.arange(padn, dtype=jnp.int32) % (NP - N))
    srcp = jnp.concatenate([edge_index[0], pad_idx])
    dstp = jnp.concatenate([edge_index[1], pad_idx])

    h_pad, sc1 = _proj(feat_p, W_fc, al, ar)
    part1 = _edge_pass1(sc1, srcp, dstp)
    nsc = _nodecalc(part1, sc1, sigma)
    wp = _edge_pass2(nsc, srcp, dstp)
    p1 = _hop(h_pad, wp, srcp, dstp)
    x1 = _comb(p1)
    p2 = _hop(x1, wp, srcp, dstp)
    x2 = _comb(p2)
    p3 = _hop(x2, wp, srcp, dstp)
    rst = _final(h_pad, x1, x2, p3, hl, hr)
    return rst[:N].reshape(N, 1, F)


# speedup vs baseline: 1.0940x; 1.0245x over previous
import functools

import numpy as np
import jax
import jax.numpy as jnp
from jax import lax
from jax.experimental import pallas as pl
from jax.experimental.pallas import tpu as pltpu
from jax.experimental.pallas import tpu_sc as plsc

N = 10000
E = 320000
F = 128
NEG = 0.2
NP = 10240
NC, NS, L = 2, 16, 16
NW = NC * NS
CH = 64
GCH = 160
EW = GCH * CH
EP = NW * EW
QUAD = GCH // 4
C10 = float(np.log(1e-10))
RPT = NP // NS

_mesh = plsc.VectorSubcoreMesh(core_axis_name="c", subcore_axis_name="s")
_sc_params = pltpu.CompilerParams(needs_layout_passes=False)


def _proj_body(feat_ref, w_ref, al_ref, ar_ref, h_ref, sc_ref):
    x = feat_ref[...]
    w = w_ref[...]
    h = lax.dot_general(x, w, (((1,), (1,)), ((), ())),
                        preferred_element_type=jnp.float32)
    h_ref[...] = h
    el = lax.dot_general(al_ref[...], h, (((1,), (1,)), ((), ())),
                         preferred_element_type=jnp.float32)
    er = lax.dot_general(ar_ref[...], h, (((1,), (1,)), ((), ())),
                         preferred_element_type=jnp.float32)
    sc_ref[...] = jnp.concatenate([el, er], axis=0)


_BA = 1024
_proj = pl.pallas_call(
    _proj_body,
    grid=(NP // _BA,),
    in_specs=[pl.BlockSpec((_BA, 128), lambda i: (i, 0)),
              pl.BlockSpec((128, 128), lambda i: (0, 0)),
              pl.BlockSpec((1, 128), lambda i: (0, 0)),
              pl.BlockSpec((1, 128), lambda i: (0, 0))],
    out_specs=[pl.BlockSpec((_BA, 128), lambda i: (i, 0)),
               pl.BlockSpec((2, _BA), lambda i: (0, i))],
    out_shape=[jax.ShapeDtypeStruct((NP, 128), jnp.float32),
               jax.ShapeDtypeStruct((2, NP), jnp.float32)],
)


@functools.partial(
    pl.kernel,
    out_type=jax.ShapeDtypeStruct((NW, 4, NP), jnp.float32),
    mesh=_mesh,
    scratch_types=[
        pltpu.VMEM((2, NP), jnp.float32),
        pltpu.VMEM((NP,), jnp.float32),
        pltpu.VMEM((NP,), jnp.float32),
        pltpu.VMEM((NP,), jnp.float32),
        pltpu.VMEM((NP,), jnp.float32),
        pltpu.VMEM((EW,), jnp.int32),
        pltpu.VMEM((EW,), jnp.int32),
    ],
    compiler_params=_sc_params,
)
def _edge_pass1(sc_hbm, src_hbm, dst_hbm, out_hbm,
                nv, a_ss, a_sd, a_do, a_di, se, de):
    cid = lax.axis_index("c")
    sid = lax.axis_index("s")
    wid = cid * NS + sid
    pltpu.sync_copy(sc_hbm, nv)
    pltpu.sync_copy(src_hbm.at[pl.ds(wid * EW, EW)], se)
    pltpu.sync_copy(dst_hbm.at[pl.ds(wid * EW, EW)], de)
    zf = jnp.zeros((16,), jnp.float32)

    def zbody(i, _):
        a_ss[pl.ds(i * 16, 16)] = zf
        a_sd[pl.ds(i * 16, 16)] = zf
        a_do[pl.ds(i * 16, 16)] = zf
        a_di[pl.ds(i * 16, 16)] = zf
        return 0

    lax.fori_loop(0, NP // 16, zbody, 0)
    c0 = jnp.zeros((16,), jnp.int32)
    c1 = jnp.ones((16,), jnp.int32)
    onef = jnp.ones((16,), jnp.float32)

    @plsc.parallel_loop(0, EW // 64, unroll=2)
    def gbody(g):
        base = g * 64
        for j in range(4):
            s = se[pl.ds(base + j * 16, 16)]
            d = de[pl.ds(base + j * 16, 16)]
            elv = plsc.load_gather(nv, [c0, s])
            erv = plsc.load_gather(nv, [c1, d])
            t = elv + erv
            ex = jnp.exp(jnp.where(t >= 0.0, t, t * NEG))
            plsc.addupdate_scatter(a_ss, [s], ex)
            plsc.addupdate_scatter(a_sd, [d], ex)
            plsc.addupdate_scatter(a_do, [s], onef)
            plsc.addupdate_scatter(a_di, [d], onef)
    pltpu.sync_copy(a_ss, out_hbm.at[wid, 0])
    pltpu.sync_copy(a_sd, out_hbm.at[wid, 1])
    pltpu.sync_copy(a_do, out_hbm.at[wid, 2])
    pltpu.sync_copy(a_di, out_hbm.at[wid, 3])


def _nodecalc_body(p_ref, sc_ref, sig_ref, o_ref):
    p = p_ref[...]
    s = jnp.sum(p, axis=0)
    el = sc_ref[0, :]
    er = sc_ref[1, :]
    ls_src = jnp.log(jnp.maximum(s[0], 1e-38))
    ls_dst = jnp.log(jnp.maximum(s[1], 1e-38))
    lo = -0.5 * jnp.log(jnp.maximum(s[2], 1.0))
    li = 0.5 * jnp.log(jnp.maximum(s[3], 1.0))
    sg = 1.0 / (1.0 + jnp.exp(-sig_ref[0]))
    sgr = jnp.full_like(el, sg)
    o_ref[...] = jnp.stack(
        [el, er, ls_src, ls_dst, lo, li, sgr, jnp.zeros_like(el)], axis=0)


_BC = 2048
_nodecalc = pl.pallas_call(
    _nodecalc_body,
    grid=(NP // _BC,),
    in_specs=[pl.BlockSpec((NW, 4, _BC), lambda i: (0, 0, i)),
              pl.BlockSpec((2, _BC), lambda i: (0, i)),
              pl.BlockSpec(memory_space=pltpu.SMEM)],
    out_specs=pl.BlockSpec((8, _BC), lambda i: (0, i)),
    out_shape=jax.ShapeDtypeStruct((8, NP), jnp.float32),
)


@functools.partial(
    pl.kernel,
    out_type=jax.ShapeDtypeStruct((EP,), jnp.float32),
    mesh=_mesh,
    scratch_types=[
        pltpu.VMEM((8, NP), jnp.float32),
        pltpu.VMEM((EW,), jnp.int32),
        pltpu.VMEM((EW,), jnp.int32),
        pltpu.VMEM((EW,), jnp.float32),
    ],
    compiler_params=_sc_params,
)
def _edge_pass2(nsc_hbm, src_hbm, dst_hbm, out_hbm, nv, se, de, wl):
    cid = lax.axis_index("c")
    sid = lax.axis_index("s")
    wid = cid * NS + sid
    pltpu.sync_copy(nsc_hbm, nv)
    pltpu.sync_copy(src_hbm.at[pl.ds(wid * EW, EW)], se)
    pltpu.sync_copy(dst_hbm.at[pl.ds(wid * EW, EW)], de)
    cs = [jnp.full((16,), k, jnp.int32) for k in range(6)]
    sgv = nv[6, pl.ds(0, 16)]

    @plsc.parallel_loop(0, EW // 16, unroll=2)
    def gbody(v):
        o = v * 16
        s = se[pl.ds(o, 16)]
        d = de[pl.ds(o, 16)]
        elv = plsc.load_gather(nv, [cs[0], s])
        erv = plsc.load_gather(nv, [cs[1], d])
        lss = plsc.load_gather(nv, [cs[2], s])
        lsd = plsc.load_gather(nv, [cs[3], d])
        lov = plsc.load_gather(nv, [cs[4], s])
        liv = plsc.load_gather(nv, [cs[5], d])
        t = elv + erv
        e = jnp.where(t >= 0.0, t, t * NEG)
        las = jnp.maximum(e - lss, C10)
        lad = jnp.maximum(e - lsd, C10)
        wl[pl.ds(o, 16)] = jnp.exp(
            sgv * lad + (1.0 - sgv) * las + lov + liv)
    pltpu.sync_copy(wl, out_hbm.at[pl.ds(wid * EW, EW)])


@functools.partial(
    pl.kernel,
    out_type=jax.ShapeDtypeStruct((NC, NP, 128), jnp.float32),
    mesh=_mesh,
    scratch_types=[
        pltpu.VMEM((EW,), jnp.int32),
        pltpu.VMEM((CH,), jnp.int32),
        pltpu.VMEM((CH,), jnp.int32),
        pltpu.VMEM((CH,), jnp.int32),
        pltpu.VMEM((CH,), jnp.int32),
        pltpu.VMEM((CH,), jnp.float32),
        pltpu.VMEM((CH,), jnp.float32),
        pltpu.VMEM((CH,), jnp.float32),
        pltpu.VMEM((CH,), jnp.float32),
        pltpu.VMEM((CH, 128), jnp.float32),
        pltpu.VMEM((CH, 128), jnp.float32),
        pltpu.VMEM((CH, 128), jnp.float32),
        pltpu.VMEM((CH, 128), jnp.float32),
        pltpu.VMEM_SHARED((NP, 128), jnp.float32),
    ] + [pltpu.SemaphoreType.DMA] * 12,
    compiler_params=_sc_params,
)
def _hop(x_hbm, w_hbm, src_hbm, dst_hbm, out_hbm,
         se, db0, db1, db2, db3, wb0, wb1, wb2, wb3,
         rows0, rows1, rows2, rows3, acc,
         semg0, semg1, semg2, semg3, semi0, semi1, semi2, semi3,
         sems0, sems1, sems2, sems3):
    cid = lax.axis_index("c")
    sid = lax.axis_index("s")
    wid = cid * NS + sid
    pltpu.sync_copy(src_hbm.at[pl.ds(wid * EW, EW)], se)
    zf = jnp.zeros((16,), jnp.float32)

    def zrow(r, _):
        for j in range(8):
            rows0[r, pl.ds(j * 16, 16)] = zf
        return 0

    lax.fori_loop(0, CH, zrow, 0)
    base0 = sid * RPT
    for b in range(RPT // CH):
        pltpu.async_copy(rows0, acc.at[pl.ds(base0 + b * CH, CH)], semi0)
    rem = RPT - (RPT // CH) * CH
    if rem:
        pltpu.async_copy(rows0.at[pl.ds(0, rem)],
                         acc.at[pl.ds(base0 + (RPT // CH) * CH, rem)], semi0)
    for b in range(RPT // CH):
        pltpu.make_async_copy(rows0, acc.at[pl.ds(base0 + b * CH, CH)],
                              semi0).wait()
    if rem:
        pltpu.make_async_copy(rows0.at[pl.ds(0, rem)],
                              acc.at[pl.ds(base0 + (RPT // CH) * CH, rem)],
                              semi0).wait()
    plsc.subcore_barrier()

    rows = (rows0, rows1, rows2, rows3)
    dbs = (db0, db1, db2, db3)
    wbs = (wb0, wb1, wb2, wb3)
    semg = (semg0, semg1, semg2, semg3)
    semi = (semi0, semi1, semi2, semi3)
    sems = (sems0, sems1, sems2, sems3)

    def prefetch(g, slot):
        pltpu.async_copy(w_hbm.at[pl.ds(wid * EW + g * CH, CH)], wbs[slot],
                         semi[slot])
        pltpu.async_copy(dst_hbm.at[pl.ds(wid * EW + g * CH, CH)], dbs[slot],
                         semi[slot])
        pltpu.async_copy(x_hbm.at[se.at[pl.ds(g * CH, CH)]], rows[slot],
                         semg[slot])

    def wait_scatter(slot):
        pltpu.make_async_copy(rows[slot], acc.at[dbs[slot]],
                              sems[slot]).wait()

    def process(g, slot):
        rb = rows[slot]
        wb = wbs[slot]
        pltpu.make_async_copy(w_hbm.at[pl.ds(wid * EW + g * CH, CH)], wb,
                              semi[slot]).wait()
        pltpu.make_async_copy(dst_hbm.at[pl.ds(wid * EW + g * CH, CH)],
                              dbs[slot], semi[slot]).wait()
        pltpu.make_async_copy(x_hbm.at[se.at[pl.ds(g * CH, CH)]], rb,
                              semg[slot]).wait()

        @plsc.parallel_loop(0, CH // 16, unroll=2)
        def sgrp(q):
            wv16 = wb[pl.ds(q * 16, 16)]
            base = q * 16
            for r in range(16):
                wvr = jnp.broadcast_to(wv16[r], (16,))
                for j in range(8):
                    rb[base + r, pl.ds(j * 16, 16)] = (
                        rb[base + r, pl.ds(j * 16, 16)] * wvr)
        pltpu.async_copy(rb, acc.at[dbs[slot]], sems[slot], add=True)

    prefetch(0, 0)
    prefetch(1, 1)

    def rbody(r, _):
        for b in range(4):
            g = 4 * r + b
            slot = b
            slotp = (b + 2) % 4
            if b <= 1:
                @pl.when(r >= 1)
                def _():
                    wait_scatter(slotp)
                prefetch(g + 2, slotp)
            else:
                @pl.when(r < QUAD - 1)
                def _():
                    wait_scatter(slotp)
                    prefetch(g + 2, slotp)
            process(g, slot)
        return 0

    lax.fori_loop(0, QUAD, rbody, 0)
    for s in range(4):
        wait_scatter(s)
    plsc.subcore_barrier()
    for b in range(RPT // 128):
        r0 = sid * RPT + b * 128
        pltpu.async_copy(acc.at[pl.ds(r0, 128)],
                         out_hbm.at[cid, pl.ds(r0, 128)], semi0)
    for b in range(RPT // 128):
        r0 = sid * RPT + b * 128
        pltpu.make_async_copy(acc.at[pl.ds(r0, 128)],
                              out_hbm.at[cid, pl.ds(r0, 128)], semi0).wait()


def _comb_body(p_ref, o_ref):
    o_ref[...] = p_ref[0] + p_ref[1]


_BB = 1024
_comb = pl.pallas_call(
    _comb_body,
    grid=(NP // _BB,),
    in_specs=[pl.BlockSpec((NC, _BB, 128), lambda i: (0, i, 0))],
    out_specs=pl.BlockSpec((_BB, 128), lambda i: (i, 0)),
    out_shape=jax.ShapeDtypeStruct((NP, 128), jnp.float32),
)


def _final_body(h_ref, x1_ref, x2_ref, p3_ref, hl_ref, hr_ref, o_ref):
    h = h_ref[...]
    x1 = x1_ref[...]
    x2 = x2_ref[...]
    x3 = p3_ref[0] + p3_ref[1]
    hl = hl_ref[...]
    hr = hr_ref[...]
    al = jnp.sum(h * hl, axis=1, keepdims=True)
    xs = (h, x1, x2, x3)
    ls = []
    for x in xs:
        v = al + jnp.sum(x * hr, axis=1, keepdims=True)
        ls.append(jnp.where(v >= 0.0, v, v * NEG))
    m = jnp.maximum(jnp.maximum(ls[0], ls[1]), jnp.maximum(ls[2], ls[3]))
    es = [jnp.exp(v - m) for v in ls]
    tot = es[0] + es[1] + es[2] + es[3]
    o_ref[...] = (h * es[0] + x1 * es[1] + x2 * es[2] + x3 * es[3]) / tot


_final = pl.pallas_call(
    _final_body,
    grid=(NP // _BA,),
    in_specs=[pl.BlockSpec((_BA, 128), lambda i: (i, 0)),
              pl.BlockSpec((_BA, 128), lambda i: (i, 0)),
              pl.BlockSpec((_BA, 128), lambda i: (i, 0)),
              pl.BlockSpec((NC, _BA, 128), lambda i: (0, i, 0)),
              pl.BlockSpec((1, 128), lambda i: (0, 0)),
              pl.BlockSpec((1, 128), lambda i: (0, 0))],
    out_specs=pl.BlockSpec((_BA, 128), lambda i: (i, 0)),
    out_shape=jax.ShapeDtypeStruct((NP, 128), jnp.float32),
)


def kernel(feat, W_fc, attn_l, attn_r, hop_attn_l, hop_attn_r, sigma,
           edge_index):
    feat_p = jnp.pad(feat, ((0, NP - N), (0, 0)))
    al = attn_l.reshape(1, F)
    ar = attn_r.reshape(1, F)
    hl = hop_attn_l.reshape(1, F)
    hr = hop_attn_r.reshape(1, F)
    padn = EP - E
    pad_idx = N + (jnp.arange(padn, dtype=jnp.int32) % (NP - N))
    srcp = jnp.concatenate([edge_index[0], pad_idx])
    dstp = jnp.concatenate([edge_index[1], pad_idx])

    h_pad, sc1 = _proj(feat_p, W_fc, al, ar)
    part1 = _edge_pass1(sc1, srcp, dstp)
    nsc = _nodecalc(part1, sc1, sigma)
    wp = _edge_pass2(nsc, srcp, dstp)
    p1 = _hop(h_pad, wp, srcp, dstp)
    x1 = _comb(p1)
    p2 = _hop(x1, wp, srcp, dstp)
    x2 = _comb(p2)
    p3 = _hop(x2, wp, srcp, dstp)
    rst = _final(h_pad, x1, x2, p3, hl, hr)
    return rst[:N].reshape(N, 1, F)
